# baseline (speedup 1.0000x reference)
import functools

import jax
import jax.numpy as jnp
from jax import lax
from jax.experimental import pallas as pl
from jax.experimental.pallas import tpu as pltpu
from jax.experimental.pallas import tpu_sc as plsc

N = 10000
E = 320000
D = 128
NC = 2
NS = 16
L = 16
NW = NC * NS
E_W = E // NW
CH = 80
NCHUNK = E_W // CH
ROWS_T = 624
TAIL_OFF = NS * ROWS_T
TAIL = N - TAIL_OFF


def _sc_agg_body(compute_deg, h_hbm, src_hbm, dst_hbm, w_hbm, z2d_hbm, z1d_hbm,
                 *refs):
    if compute_deg:
        agg_out, deg_out = refs[0], refs[1]
        scratches = refs[2:]
    else:
        agg_out = refs[0]
        scratches = refs[1:]
    (agg_s, deg_s, src_a, dst_a, w_a, rows_a, src_b, dst_b, w_b, rows_b,
     ones_v, sem) = scratches

    c = lax.axis_index("c")
    s = lax.axis_index("s")
    wid = c * NS + s

    pltpu.sync_copy(z2d_hbm, agg_s.at[pl.ds(s * ROWS_T, ROWS_T)])

    @pl.when(s == NS - 1)
    def _():
        pltpu.sync_copy(z2d_hbm.at[pl.ds(0, TAIL)], agg_s.at[pl.ds(TAIL_OFF, TAIL)])

    if compute_deg:
        @pl.when(s == 0)
        def _():
            pltpu.sync_copy(z1d_hbm, deg_s)
        for g in range(CH // L):
            ones_v[pl.ds(g * L, L)] = jnp.ones((L,), jnp.float32)
    iota = lax.iota(jnp.int32, L)
    plsc.subcore_barrier()

    base = wid * E_W

    def do_chunk(i, src_v, dst_v, w_v, rows_v):
        off = base + i * CH
        pltpu.sync_copy(src_hbm.at[pl.ds(off, CH)], src_v)
        pltpu.sync_copy(dst_hbm.at[pl.ds(off, CH)], dst_v)
        pltpu.sync_copy(w_hbm.at[pl.ds(off, CH)], w_v)
        for g in range(CH // L):
            i16 = src_v[pl.ds(g * L, L)]
            pltpu.async_copy(h_hbm.at[i16], rows_v.at[pl.ds(g * L, L)], sem).wait()
        for g in range(CH // L):
            w16 = w_v[pl.ds(g * L, L)]
            for j in range(L):
                wb = w16[jnp.full((L,), j, jnp.int32)]
                e = g * L + j
                for q in range(D // L):
                    sl = pl.ds(q * L, L)
                    rows_v[e, sl] = rows_v[e, sl] * wb
        pltpu.sync_copy(rows_v, agg_s.at[dst_v], add=True)
        if compute_deg:
            pltpu.sync_copy(ones_v, deg_s.at[dst_v], add=True)

    def pair_body(p, carry):
        do_chunk(2 * p, src_a, dst_a, w_a, rows_a)
        do_chunk(2 * p + 1, src_b, dst_b, w_b, rows_b)
        return carry

    lax.fori_loop(0, NCHUNK // 2, pair_body, 0)
    if NCHUNK % 2:
        do_chunk(NCHUNK - 1, src_a, dst_a, w_a, rows_a)
    plsc.subcore_barrier()

    pltpu.sync_copy(agg_s.at[pl.ds(s * ROWS_T, ROWS_T)],
                    agg_out.at[c, pl.ds(s * ROWS_T, ROWS_T)])

    @pl.when(s == NS - 1)
    def _():
        pltpu.sync_copy(agg_s.at[pl.ds(TAIL_OFF, TAIL)],
                        agg_out.at[c, pl.ds(TAIL_OFF, TAIL)])

    if compute_deg:
        @pl.when(s == 0)
        def _():
            pltpu.sync_copy(deg_s, deg_out.at[c, 0])


def _make_sc_aggregate(compute_deg):
    out_type = [jax.ShapeDtypeStruct((NC, N, D), jnp.float32)]
    if compute_deg:
        out_type.append(jax.ShapeDtypeStruct((NC, 1, N), jnp.float32))
    mesh = plsc.VectorSubcoreMesh(core_axis_name="c", subcore_axis_name="s")
    return pl.kernel(
        functools.partial(_sc_agg_body, compute_deg),
        out_type=out_type,
        mesh=mesh,
        scratch_types=[
            pltpu.VMEM_SHARED((N, D), jnp.float32),
            pltpu.VMEM_SHARED((N,), jnp.float32),
            pltpu.VMEM((CH,), jnp.int32),
            pltpu.VMEM((CH,), jnp.int32),
            pltpu.VMEM((CH,), jnp.float32),
            pltpu.VMEM((CH, D), jnp.float32),
            pltpu.VMEM((CH,), jnp.int32),
            pltpu.VMEM((CH,), jnp.int32),
            pltpu.VMEM((CH,), jnp.float32),
            pltpu.VMEM((CH, D), jnp.float32),
            pltpu.VMEM((CH,), jnp.float32),
            pltpu.SemaphoreType.DMA,
        ],
        compiler_params=pltpu.CompilerParams(needs_layout_passes=False),
        name="sage_sc_aggregate" + ("_deg" if compute_deg else ""),
    )


_sc_aggregate_deg = _make_sc_aggregate(True)
_sc_aggregate = _make_sc_aggregate(False)

BM = 2000
GRID = N // BM


def _tc_layer_body(relu, x_ref, agg_ref, deg_ref, ws_ref, wn_ref, b_ref, o_ref):
    deg = jnp.sum(deg_ref[...], axis=(0, 1))
    agg = agg_ref[0] + agg_ref[1]
    hn = agg / jnp.clip(deg, 1.0, None)[:, None]
    out = (jnp.dot(x_ref[...], ws_ref[...], preferred_element_type=jnp.float32,
                   precision=jax.lax.Precision.HIGHEST)
           + jnp.dot(hn, wn_ref[...], preferred_element_type=jnp.float32,
                     precision=jax.lax.Precision.HIGHEST)
           + b_ref[...])
    o_ref[...] = jnp.maximum(out, 0.0) if relu else out


def _tc_layer(x, agg, deg, W_self, W_neigh, b, relu):
    deg_r = jnp.transpose(deg.reshape(NC, GRID, BM), (1, 0, 2))
    b2 = b.reshape(1, D)
    return pl.pallas_call(
        functools.partial(_tc_layer_body, relu),
        grid=(GRID,),
        in_specs=[
            pl.BlockSpec((BM, D), lambda i: (i, 0)),
            pl.BlockSpec((NC, BM, D), lambda i: (0, i, 0)),
            pl.BlockSpec((1, NC, BM), lambda i: (i, 0, 0)),
            pl.BlockSpec((D, D), lambda i: (0, 0)),
            pl.BlockSpec((D, D), lambda i: (0, 0)),
            pl.BlockSpec((1, D), lambda i: (0, 0)),
        ],
        out_specs=pl.BlockSpec((BM, D), lambda i: (i, 0)),
        out_shape=jax.ShapeDtypeStruct((N, D), jnp.float32),
        name="sage_tc_layer",
    )(x, agg, deg_r, W_self, W_neigh, b2)


def kernel(inputs, edge_index, edge_weight, W_self1, W_neigh1, b1,
           W_self2, W_neigh2, b2):
    x = inputs
    src = edge_index[0].astype(jnp.int32)
    dst = edge_index[1].astype(jnp.int32)
    w = edge_weight.astype(jnp.float32)
    z2d = jnp.zeros((ROWS_T, D), jnp.float32)
    z1d = jnp.zeros((N,), jnp.float32)

    agg1, deg = _sc_aggregate_deg(x, src, dst, w, z2d, z1d)
    deg = deg[:, 0, :]
    h1 = _tc_layer(x, agg1, deg, W_self1, W_neigh1, b1, relu=True)
    (agg2,) = _sc_aggregate(h1, src, dst, w, z2d, z1d)
    out = _tc_layer(h1, agg2, deg, W_self2, W_neigh2, b2, relu=False)
    return out


# speedup vs baseline: 1.6213x; 1.6213x over previous
import functools

import jax
import jax.numpy as jnp
from jax import lax
from jax.experimental import pallas as pl
from jax.experimental.pallas import tpu as pltpu
from jax.experimental.pallas import tpu_sc as plsc

N = 10000
E = 320000
D = 128
NC = 2
NS = 16
L = 16
NW = NC * NS
E_W = E // NW
CH = 80
NCHUNK = E_W // CH
ROWS_T = 624
TAIL_OFF = NS * ROWS_T
TAIL = N - TAIL_OFF


def _sc_agg_body(compute_deg, h_hbm, src_hbm, dst_hbm, w_hbm, z2d_hbm, z1d_hbm,
                 *refs):
    if compute_deg:
        agg_out, deg_out = refs[0], refs[1]
        scratches = refs[2:]
    else:
        agg_out = refs[0]
        scratches = refs[1:]
    (agg_s, deg_s, src_a, dst_a, w_a, rows_a, src_b, dst_b, w_b, rows_b,
     ones_v, sem) = scratches

    c = lax.axis_index("c")
    s = lax.axis_index("s")
    wid = c * NS + s

    pltpu.sync_copy(z2d_hbm, agg_s.at[pl.ds(s * ROWS_T, ROWS_T)])

    @pl.when(s == NS - 1)
    def _():
        pltpu.sync_copy(z2d_hbm.at[pl.ds(0, TAIL)], agg_s.at[pl.ds(TAIL_OFF, TAIL)])

    if compute_deg:
        @pl.when(s == 0)
        def _():
            pltpu.sync_copy(z1d_hbm, deg_s)
        for g in range(CH // L):
            ones_v[pl.ds(g * L, L)] = jnp.ones((L,), jnp.float32)
    iota = lax.iota(jnp.int32, L)
    plsc.subcore_barrier()

    base = wid * E_W

    def do_chunk(i, src_v, dst_v, w_v, rows_v):
        off = base + i * CH
        pltpu.sync_copy(src_hbm.at[pl.ds(off, CH)], src_v)
        pltpu.sync_copy(dst_hbm.at[pl.ds(off, CH)], dst_v)
        pltpu.sync_copy(w_hbm.at[pl.ds(off, CH)], w_v)
        descs = [
            pltpu.async_copy(h_hbm.at[src_v[pl.ds(g * L, L)]],
                             rows_v.at[pl.ds(g * L, L)], sem)
            for g in range(CH // L)
        ]
        for d in descs:
            d.wait()
        for g in range(CH // L):
            w16 = w_v[pl.ds(g * L, L)]
            for j in range(L):
                wb = w16[jnp.full((L,), j, jnp.int32)]
                e = g * L + j
                for q in range(D // L):
                    sl = pl.ds(q * L, L)
                    rows_v[e, sl] = rows_v[e, sl] * wb
        pltpu.sync_copy(rows_v, agg_s.at[dst_v], add=True)
        if compute_deg:
            pltpu.sync_copy(ones_v, deg_s.at[dst_v], add=True)

    def pair_body(p, carry):
        do_chunk(2 * p, src_a, dst_a, w_a, rows_a)
        do_chunk(2 * p + 1, src_b, dst_b, w_b, rows_b)
        return carry

    lax.fori_loop(0, NCHUNK // 2, pair_body, 0)
    if NCHUNK % 2:
        do_chunk(NCHUNK - 1, src_a, dst_a, w_a, rows_a)
    plsc.subcore_barrier()

    pltpu.sync_copy(agg_s.at[pl.ds(s * ROWS_T, ROWS_T)],
                    agg_out.at[c, pl.ds(s * ROWS_T, ROWS_T)])

    @pl.when(s == NS - 1)
    def _():
        pltpu.sync_copy(agg_s.at[pl.ds(TAIL_OFF, TAIL)],
                        agg_out.at[c, pl.ds(TAIL_OFF, TAIL)])

    if compute_deg:
        @pl.when(s == 0)
        def _():
            pltpu.sync_copy(deg_s, deg_out.at[c, 0])


def _make_sc_aggregate(compute_deg):
    out_type = [jax.ShapeDtypeStruct((NC, N, D), jnp.float32)]
    if compute_deg:
        out_type.append(jax.ShapeDtypeStruct((NC, 1, N), jnp.float32))
    mesh = plsc.VectorSubcoreMesh(core_axis_name="c", subcore_axis_name="s")
    return pl.kernel(
        functools.partial(_sc_agg_body, compute_deg),
        out_type=out_type,
        mesh=mesh,
        scratch_types=[
            pltpu.VMEM_SHARED((N, D), jnp.float32),
            pltpu.VMEM_SHARED((N,), jnp.float32),
            pltpu.VMEM((CH,), jnp.int32),
            pltpu.VMEM((CH,), jnp.int32),
            pltpu.VMEM((CH,), jnp.float32),
            pltpu.VMEM((CH, D), jnp.float32),
            pltpu.VMEM((CH,), jnp.int32),
            pltpu.VMEM((CH,), jnp.int32),
            pltpu.VMEM((CH,), jnp.float32),
            pltpu.VMEM((CH, D), jnp.float32),
            pltpu.VMEM((CH,), jnp.float32),
            pltpu.SemaphoreType.DMA,
        ],
        compiler_params=pltpu.CompilerParams(needs_layout_passes=False),
        name="sage_sc_aggregate" + ("_deg" if compute_deg else ""),
    )


_sc_aggregate_deg = _make_sc_aggregate(True)
_sc_aggregate = _make_sc_aggregate(False)

BM = 2000
GRID = N // BM


def _tc_layer_body(relu, x_ref, agg_ref, deg_ref, ws_ref, wn_ref, b_ref, o_ref):
    deg = jnp.sum(deg_ref[...], axis=(0, 1))
    agg = agg_ref[0] + agg_ref[1]
    hn = agg / jnp.clip(deg, 1.0, None)[:, None]
    out = (jnp.dot(x_ref[...], ws_ref[...], preferred_element_type=jnp.float32,
                   precision=jax.lax.Precision.HIGHEST)
           + jnp.dot(hn, wn_ref[...], preferred_element_type=jnp.float32,
                     precision=jax.lax.Precision.HIGHEST)
           + b_ref[...])
    o_ref[...] = jnp.maximum(out, 0.0) if relu else out


def _tc_layer(x, agg, deg, W_self, W_neigh, b, relu):
    deg_r = jnp.transpose(deg.reshape(NC, GRID, BM), (1, 0, 2))
    b2 = b.reshape(1, D)
    return pl.pallas_call(
        functools.partial(_tc_layer_body, relu),
        grid=(GRID,),
        in_specs=[
            pl.BlockSpec((BM, D), lambda i: (i, 0)),
            pl.BlockSpec((NC, BM, D), lambda i: (0, i, 0)),
            pl.BlockSpec((1, NC, BM), lambda i: (i, 0, 0)),
            pl.BlockSpec((D, D), lambda i: (0, 0)),
            pl.BlockSpec((D, D), lambda i: (0, 0)),
            pl.BlockSpec((1, D), lambda i: (0, 0)),
        ],
        out_specs=pl.BlockSpec((BM, D), lambda i: (i, 0)),
        out_shape=jax.ShapeDtypeStruct((N, D), jnp.float32),
        name="sage_tc_layer",
    )(x, agg, deg_r, W_self, W_neigh, b2)


def kernel(inputs, edge_index, edge_weight, W_self1, W_neigh1, b1,
           W_self2, W_neigh2, b2):
    x = inputs
    src = edge_index[0].astype(jnp.int32)
    dst = edge_index[1].astype(jnp.int32)
    w = edge_weight.astype(jnp.float32)
    z2d = jnp.zeros((ROWS_T, D), jnp.float32)
    z1d = jnp.zeros((N,), jnp.float32)

    agg1, deg = _sc_aggregate_deg(x, src, dst, w, z2d, z1d)
    deg = deg[:, 0, :]
    h1 = _tc_layer(x, agg1, deg, W_self1, W_neigh1, b1, relu=True)
    (agg2,) = _sc_aggregate(h1, src, dst, w, z2d, z1d)
    out = _tc_layer(h1, agg2, deg, W_self2, W_neigh2, b2, relu=False)
    return out
